# baseline (speedup 1.0000x reference)
import functools

import jax
import jax.numpy as jnp
from jax import lax
from jax.experimental import pallas as pl
from jax.experimental.pallas import tpu as pltpu
from jax.experimental.pallas import tpu_sc as plsc

VOCAB = 1000000
EMBED_DIM = 64
BATCH = 16384
L = 16
RND = 256


@functools.partial(jax.jit, static_argnames=("num_workers",))
def _sc_embed_dot(x0h, x1h, x0p, x1p, w2, *, num_workers):
    bpw = BATCH // num_workers
    mesh = plsc.VectorSubcoreMesh(core_axis_name="c", subcore_axis_name="s")
    num_cores = mesh.num_cores

    @functools.partial(
        pl.kernel,
        out_type=jax.ShapeDtypeStruct((BATCH,), jnp.float32),
        mesh=mesh,
        scratch_types=[
            pltpu.VMEM((bpw,), jnp.int32),
            pltpu.VMEM((bpw,), jnp.int32),
            pltpu.VMEM((bpw,), jnp.int32),
            pltpu.VMEM((bpw,), jnp.int32),
            pltpu.VMEM((RND, 2 * EMBED_DIM), jnp.float32),
            pltpu.VMEM((RND, 2 * EMBED_DIM), jnp.float32),
            pltpu.VMEM((bpw,), jnp.float32),
            pltpu.SemaphoreType.DMA,
            pltpu.SemaphoreType.DMA,
        ],
        compiler_params=pltpu.CompilerParams(needs_layout_passes=False),
    )
    def k(x0h_hbm, x1h_hbm, x0p_hbm, x1p_hbm, w2_hbm, out_hbm,
          i0_v, i1_v, p0_v, p1_v, e0_v, e1_v, out_v, sem0, sem1):
        wid = lax.axis_index("s") * num_cores + lax.axis_index("c")
        base = wid * bpw
        pltpu.sync_copy(x0h_hbm.at[pl.ds(base, bpw)], i0_v)
        pltpu.sync_copy(x1h_hbm.at[pl.ds(base, bpw)], i1_v)
        pltpu.sync_copy(x0p_hbm.at[pl.ds(base, bpw)], p0_v)
        pltpu.sync_copy(x1p_hbm.at[pl.ds(base, bpw)], p1_v)

        for r in range(bpw // RND):
            r0 = r * RND
            c0 = pltpu.async_copy(
                w2_hbm.at[i0_v.at[pl.ds(r0, RND)]], e0_v, sem0)
            c1 = pltpu.async_copy(
                w2_hbm.at[i1_v.at[pl.ds(r0, RND)]], e1_v, sem1)
            c0.wait()
            c1.wait()

            def group(g, _, r0=r0):
                o = g * L
                rows = lax.iota(jnp.int32, L) + o
                col0 = p0_v[pl.ds(r0 + o, L)] * EMBED_DIM
                col1 = p1_v[pl.ds(r0 + o, L)] * EMBED_DIM
                acc = jnp.zeros((L,), jnp.float32)
                for d in range(EMBED_DIM):
                    a = plsc.load_gather(e0_v, [rows, col0 + d])
                    b = plsc.load_gather(e1_v, [rows, col1 + d])
                    acc = acc + a * b
                out_v[pl.ds(r0 + o, L)] = 1.0 / (1.0 + jnp.exp(-acc))
                return _

            lax.fori_loop(0, RND // L, group, 0)

        pltpu.sync_copy(out_v, out_hbm.at[pl.ds(base, bpw)])

    return k(x0h, x1h, x0p, x1p, w2)


def kernel(x, W_g):
    info = plsc.get_sparse_core_info()
    num_workers = info.num_cores * info.num_subcores
    w2 = W_g.reshape(VOCAB // 2, 2 * EMBED_DIM)
    x0, x1 = x[:, 0], x[:, 1]
    out = _sc_embed_dot(x0 >> 1, x1 >> 1, x0 & 1, x1 & 1, w2,
                        num_workers=num_workers)
    return out.reshape(BATCH, 1)
